# baseline (speedup 1.0000x reference)
import functools

import jax
import jax.numpy as jnp
from jax import lax
from jax.experimental import pallas as pl
from jax.experimental.pallas import tpu as pltpu
from jax.experimental.pallas import tpu_sc as plsc

N = 10000
E = 320000
D = 128
OUT = 128

NC = 2
NS = 16
NW = NC * NS
B = 128
NB = 79
EW = NB * B
E_PAD = NW * EW
ZT = 1000
ZNT = N // ZT
LANES = 16

_mesh = plsc.VectorSubcoreMesh(core_axis_name="c", subcore_axis_name="s")


@functools.partial(
    pl.kernel,
    out_type=jax.ShapeDtypeStruct((NC, N, D), jnp.float32),
    mesh=_mesh,
    compiler_params=pltpu.CompilerParams(use_tc_tiling_on_sc=False,
                                         needs_layout_passes=False),
    scratch_types=[
        pltpu.VMEM((NB, B), jnp.int32),
        pltpu.VMEM((NB, B), jnp.int32),
        pltpu.VMEM((NB, B // 2), jnp.int32),
        pltpu.VMEM((B, D // 2), jnp.int32),
        pltpu.VMEM((B, D), jnp.float32),
        pltpu.VMEM_SHARED((N, D), jnp.float32),
        pltpu.SemaphoreType.DMA,
    ],
)
def _sc_aggregate(x_hbm, src_hbm, dst_hbm, adj_hbm, zeros_hbm, out_hbm,
                  src_v, dst_v, adj_v, rows_v, sbuf, acc, sem):
    c = lax.axis_index("c")
    s = lax.axis_index("s")
    wid = s * NC + c

    @pl.when(s < ZNT)
    def _zero():
        pltpu.sync_copy(zeros_hbm, acc.at[pl.ds(s * ZT, ZT)])

    pltpu.sync_copy(src_hbm.at[wid], src_v)
    pltpu.sync_copy(dst_hbm.at[wid], dst_v)
    pltpu.sync_copy(adj_hbm.at[wid], adj_v)
    plsc.subcore_barrier()

    def block(b, carry):
        pltpu.async_copy(x_hbm.at[src_v.at[b]], rows_v, sem).wait()

        hi_mask = jnp.int32(-65536)

        def rowgrp(g, carry2):
            wv = adj_v[b, pl.ds(g * LANES, LANES)]
            wlo = plsc.bitcast(lax.shift_left(wv, 16), jnp.float32)
            whi = plsc.bitcast(lax.bitwise_and(wv, hi_mask), jnp.float32)
            wab = (wlo, whi)
            for k in range(LANES):
                for h in range(2):
                    i = g * 2 * LANES + 2 * k + h
                    w = wab[h][k]
                    for j in range(D // (2 * LANES)):
                        v = rows_v[i, pl.ds(j * LANES, LANES)]
                        a = plsc.bitcast(lax.shift_left(v, 16), jnp.float32)
                        b2 = plsc.bitcast(lax.bitwise_and(v, hi_mask),
                                          jnp.float32)
                        base = j * 2 * LANES
                        sbuf[i, pl.ds(base, LANES)] = a * w
                        sbuf[i, pl.ds(base + LANES, LANES)] = b2 * w
            return carry2

        lax.fori_loop(0, B // (2 * LANES), rowgrp, 0)

        pltpu.sync_copy(sbuf, acc.at[dst_v.at[b]], add=True)
        return carry

    lax.fori_loop(0, NB, block, 0)
    plsc.subcore_barrier()

    @pl.when(s < ZNT)
    def _copy_out():
        pltpu.sync_copy(acc.at[pl.ds(s * ZT, ZT)],
                        out_hbm.at[c, pl.ds(s * ZT, ZT)])


_ROWS_BLK = 1000


def _tc_finish(p_ref, w_ref, o_ref):
    ssum = p_ref[0] + p_ref[1]
    o_ref[...] = jnp.maximum(
        jnp.dot(ssum, w_ref[...], preferred_element_type=jnp.float32), 0.0)


@jax.jit
def kernel(x, edge_index, adj_vals, W):
    ei = edge_index.astype(jnp.int32)
    pad = E_PAD - E
    src = jnp.concatenate([ei[0], jnp.zeros((pad,), jnp.int32)])
    dst = jnp.concatenate([ei[1], jnp.zeros((pad,), jnp.int32)])
    adj = jnp.concatenate([adj_vals, jnp.zeros((pad,), jnp.float32)])
    src = src.reshape(NW, NB, B)
    dst = dst.reshape(NW, NB, B)
    adjb = jax.lax.bitcast_convert_type(
        adj.astype(jnp.bfloat16).reshape(NW, NB, B // 2, 2), jnp.int32)
    xsw = x.reshape(N, D // 32, 2, LANES).transpose(0, 1, 3, 2).reshape(N, D)
    xb = jax.lax.bitcast_convert_type(
        xsw.astype(jnp.bfloat16).reshape(N, D // 2, 2), jnp.int32)
    zeros = jnp.zeros((ZT, D), jnp.float32)

    partials = _sc_aggregate(xb, src, dst, adjb, zeros)

    out = pl.pallas_call(
        _tc_finish,
        grid=(N // _ROWS_BLK,),
        in_specs=[
            pl.BlockSpec((NC, _ROWS_BLK, D), lambda i: (0, i, 0)),
            pl.BlockSpec((D, OUT), lambda i: (0, 0)),
        ],
        out_specs=pl.BlockSpec((_ROWS_BLK, OUT), lambda i: (i, 0)),
        out_shape=jax.ShapeDtypeStruct((N, OUT), jnp.float32),
    )(partials, W)
    return out


# speedup vs baseline: 1.4257x; 1.4257x over previous
import functools

import jax
import jax.numpy as jnp
from jax import lax
from jax.experimental import pallas as pl
from jax.experimental.pallas import tpu as pltpu
from jax.experimental.pallas import tpu_sc as plsc

N = 10000
E = 320000
D = 128
OUT = 128

NC = 2
NS = 16
NW = NC * NS
B = 128
NB = 79
EW = NB * B
E_PAD = NW * EW
ZT = 1000
ZNT = N // ZT
LANES = 16

_mesh = plsc.VectorSubcoreMesh(core_axis_name="c", subcore_axis_name="s")


@functools.partial(
    pl.kernel,
    out_type=jax.ShapeDtypeStruct((NC, N, D), jnp.float32),
    mesh=_mesh,
    compiler_params=pltpu.CompilerParams(use_tc_tiling_on_sc=False),
    scratch_types=[
        pltpu.VMEM((NB, B), jnp.int32),
        pltpu.VMEM((NB, B), jnp.int32),
        pltpu.VMEM((NB, B), jnp.float32),
        pltpu.VMEM((B, D), jnp.float32),
        pltpu.VMEM_SHARED((N, D), jnp.float32),
        pltpu.SemaphoreType.DMA,
    ],
)
def _sc_aggregate(x_hbm, src_hbm, dst_hbm, adj_hbm, zeros_hbm, out_hbm,
                  src_v, dst_v, adj_v, rows_v, acc, sem):
    c = lax.axis_index("c")
    s = lax.axis_index("s")
    wid = s * NC + c

    @pl.when(s < ZNT)
    def _zero():
        pltpu.sync_copy(zeros_hbm, acc.at[pl.ds(s * ZT, ZT)])

    pltpu.sync_copy(src_hbm.at[wid], src_v)
    pltpu.sync_copy(dst_hbm.at[wid], dst_v)
    pltpu.sync_copy(adj_hbm.at[wid], adj_v)
    plsc.subcore_barrier()

    def block(b, carry):
        pltpu.async_copy(x_hbm.at[src_v.at[b]], rows_v, sem).wait()

        def rowgrp(g, carry2):
            wvec = adj_v[b, pl.ds(g * LANES, LANES)]
            for k in range(LANES):
                i = g * LANES + k
                w = wvec[k]
                for j in range(D // LANES):
                    sl = pl.ds(j * LANES, LANES)
                    rows_v[i, sl] = rows_v[i, sl] * w
            return carry2

        lax.fori_loop(0, B // LANES, rowgrp, 0)

        pltpu.sync_copy(rows_v, acc.at[dst_v.at[b]], add=True)
        return carry

    lax.fori_loop(0, NB, block, 0)
    plsc.subcore_barrier()

    @pl.when(s < ZNT)
    def _copy_out():
        pltpu.sync_copy(acc.at[pl.ds(s * ZT, ZT)],
                        out_hbm.at[c, pl.ds(s * ZT, ZT)])


_ROWS_BLK = 1000


def _tc_finish(p_ref, w_ref, o_ref):
    ssum = p_ref[0] + p_ref[1]
    o_ref[...] = jnp.maximum(
        jnp.dot(ssum, w_ref[...], preferred_element_type=jnp.float32), 0.0)


@jax.jit
def kernel(x, edge_index, adj_vals, W):
    ei = edge_index.astype(jnp.int32)
    pad = E_PAD - E
    src = jnp.concatenate([ei[0], jnp.zeros((pad,), jnp.int32)])
    dst = jnp.concatenate([ei[1], jnp.zeros((pad,), jnp.int32)])
    adj = jnp.concatenate([adj_vals, jnp.zeros((pad,), jnp.float32)])
    src = src.reshape(NW, NB, B)
    dst = dst.reshape(NW, NB, B)
    adj = adj.reshape(NW, NB, B)
    zeros = jnp.zeros((ZT, D), jnp.float32)

    partials = _sc_aggregate(x, src, dst, adj, zeros)

    out = pl.pallas_call(
        _tc_finish,
        grid=(N // _ROWS_BLK,),
        in_specs=[
            pl.BlockSpec((NC, _ROWS_BLK, D), lambda i: (0, i, 0)),
            pl.BlockSpec((D, OUT), lambda i: (0, 0)),
        ],
        out_specs=pl.BlockSpec((_ROWS_BLK, OUT), lambda i: (i, 0)),
        out_shape=jax.ShapeDtypeStruct((N, OUT), jnp.float32),
    )(partials, W)
    return out
